# baseline (speedup 1.0000x reference)
import jax
import jax.numpy as jnp
from jax import lax
from jax.experimental import pallas as pl
from jax.experimental.pallas import tpu as pltpu
from jax.experimental.pallas import tpu_sc as plsc

N = 100000
NUM_FEATURES = 9
VOCAB = 128
D = 128
PAIRS = D // 2
ROWS = NUM_FEATURES * VOCAB
N_TILES = 32
ATOMS_PER_TILE = 3136
CHUNK = 64
N_PAD = 100096
FEAT_PAD = 16
LAST_FULL_CHUNKS = 43
TAIL_ATOMS = 32
FWORDS = CHUNK * FEAT_PAD
OWORDS = CHUNK * D
IDX = CHUNK * NUM_FEATURES
SUB = 96
NSUB = IDX // SUB

SCALE = 1.0 / 3.0


def _build_idx(fbuf, ibuf, natoms, iota, rowoff, m9):
    def atom_body(a, carry):
        fv = fbuf[pl.ds(a * FEAT_PAD, FEAT_PAD)] + rowoff
        plsc.store_scatter(ibuf, [jnp.full((16,), 9, jnp.int32) * a + iota],
                           fv, mask=m9)
        return carry

    lax.fori_loop(0, natoms, atom_body, 0)


def _compute_chunk(stage, obuf, natoms, iota2):
    def atom_body(k, carry):
        for half in range(2):
            a = k * 2 + half
            sbase = a * NUM_FEATURES
            obase = a * D
            for c in range(4):
                g = [plsc.bitcast(
                        stage[sbase + i, pl.ds(c * 16, 16)],
                        jnp.bfloat16) for i in range(9)]
                t01, t23 = g[0] + g[1], g[2] + g[3]
                t45, t67 = g[4] + g[5], g[6] + g[7]
                acc = ((t01 + t23) + (t45 + t67)) + g[8]
                acc32 = plsc.bitcast(acc, jnp.int32)
                even = plsc.bitcast(
                    lax.shift_left(acc32, 16), jnp.float32) * SCALE
                odd = plsc.bitcast(
                    lax.bitwise_and(acc32, jnp.int32(-65536)),
                    jnp.float32) * SCALE
                oseg = obuf.at[pl.ds(obase + c * 32, 32)]
                plsc.store_scatter(oseg, [iota2], even)
                plsc.store_scatter(oseg, [iota2 + 1], odd)
        return carry

    lax.fori_loop(0, natoms // 2, atom_body, 0)


def _body(feats_hbm, tables_hbm, out_hbm,
          stage_v, feats_v, out_v, idx_v, fsem, gsem, osem):
    t = lax.axis_index("s") * 2 + lax.axis_index("c")

    iota = lax.broadcasted_iota(jnp.int32, (16,), 0)
    iota2 = iota * 2
    rowoff = iota * VOCAB
    m9 = iota < NUM_FEATURES
    base = t * ATOMS_PER_TILE
    nchunks = jnp.where(t == N_TILES - 1, LAST_FULL_CHUNKS,
                        ATOMS_PER_TILE // CHUNK)

    def feats_dma(ci, b):
        row0 = base + ci * CHUNK
        return pltpu.make_async_copy(
            feats_hbm.at[pl.ds(row0 * FEAT_PAD, FWORDS)],
            feats_v.at[pl.ds(b * FWORDS, FWORDS)], fsem.at[b])

    def gather_dma(b, j):
        return pltpu.make_async_copy(
            tables_hbm.at[idx_v.at[pl.ds(b * IDX + j * SUB, SUB)]],
            stage_v.at[pl.ds(b * IDX + j * SUB, SUB), :], gsem.at[b])

    def out_dma(ci, b):
        return pltpu.make_async_copy(
            out_v.at[pl.ds(b * OWORDS, OWORDS)],
            out_hbm.at[pl.ds((base + ci * CHUNK) * D, OWORDS)], osem.at[b])

    def stage_chunk(ci, b):
        feats_dma(ci, b).wait()
        _build_idx(feats_v.at[pl.ds(b * FWORDS, FWORDS)],
                   idx_v.at[pl.ds(b * IDX, IDX)], CHUNK, iota, rowoff, m9)
        for j in range(NSUB):
            gather_dma(b, j).start()

    feats_dma(0, 0).start()
    stage_chunk(0, 0)
    feats_dma(1, 1).start()

    def loop_body(ci, carry):
        b = lax.rem(ci, 2)
        pb = 1 - b
        stage_chunk(ci, b)

        @pl.when(ci + 1 < nchunks)
        def _pf():
            feats_dma(ci + 1, pb).start()

        for j in range(NSUB):
            gather_dma(pb, j).wait()

        @pl.when(ci >= 3)
        def _drain():
            out_dma(ci - 3, pb).wait()

        _compute_chunk(stage_v.at[pl.ds(pb * IDX, IDX), :],
                       out_v.at[pl.ds(pb * OWORDS, OWORDS)], CHUNK, iota2)
        out_dma(ci - 1, pb).start()
        return carry

    lax.fori_loop(1, nchunks, loop_body, 0)

    lb = lax.rem(nchunks - 1, 2)
    for j in range(NSUB):
        gather_dma(lb, j).wait()

    @pl.when(nchunks >= 3)
    def _drain_last():
        out_dma(nchunks - 3, lb).wait()

    _compute_chunk(stage_v.at[pl.ds(lb * IDX, IDX), :],
                   out_v.at[pl.ds(lb * OWORDS, OWORDS)], CHUNK, iota2)
    out_dma(nchunks - 1, lb).start()
    out_dma(nchunks - 2, 1 - lb).wait()
    out_dma(nchunks - 1, lb).wait()

    @pl.when(t == N_TILES - 1)
    def _tail():
        row0 = base + LAST_FULL_CHUNKS * CHUNK
        pltpu.sync_copy(
            feats_hbm.at[pl.ds(row0 * FEAT_PAD, TAIL_ATOMS * FEAT_PAD)],
            feats_v.at[pl.ds(0, TAIL_ATOMS * FEAT_PAD)])
        _build_idx(feats_v.at[pl.ds(0, TAIL_ATOMS * FEAT_PAD)],
                   idx_v.at[pl.ds(0, TAIL_ATOMS * NUM_FEATURES)],
                   TAIL_ATOMS, iota, rowoff, m9)
        nsub_t = TAIL_ATOMS * NUM_FEATURES // SUB
        for j in range(nsub_t):
            gather_dma(0, j).start()
        for j in range(nsub_t):
            gather_dma(0, j).wait()
        _compute_chunk(stage_v.at[pl.ds(0, IDX), :],
                       out_v.at[pl.ds(0, OWORDS)], TAIL_ATOMS, iota2)
        pltpu.sync_copy(
            out_v.at[pl.ds(0, TAIL_ATOMS * D)],
            out_hbm.at[pl.ds(row0 * D, TAIL_ATOMS * D)])


@jax.jit
def kernel(feats, tables):
    feats_p = jnp.pad(
        feats, ((0, N_PAD - N), (0, FEAT_PAD - NUM_FEATURES))).reshape(-1)
    tbl = tables.astype(jnp.bfloat16).reshape(ROWS, PAIRS, 2)
    tbl = lax.bitcast_convert_type(tbl, jnp.int32)
    run = pl.kernel(
        _body,
        out_type=jax.ShapeDtypeStruct((N * D,), jnp.float32),
        mesh=plsc.VectorSubcoreMesh(
            core_axis_name="c", subcore_axis_name="s",
            num_cores=2, num_subcores=16),
        compiler_params=pltpu.CompilerParams(
            needs_layout_passes=False, use_tc_tiling_on_sc=False),
        scratch_types=[
            pltpu.VMEM((2 * IDX, PAIRS), jnp.int32),
            pltpu.VMEM((2 * FWORDS,), jnp.int32),
            pltpu.VMEM((2 * OWORDS,), jnp.float32),
            pltpu.VMEM((2 * IDX,), jnp.int32),
            pltpu.SemaphoreType.DMA((2,)),
            pltpu.SemaphoreType.DMA((2,)),
            pltpu.SemaphoreType.DMA((2,)),
        ],
    )
    return run(feats_p, tbl).reshape(N, D)


# speedup vs baseline: 1.1179x; 1.1179x over previous
import jax
import jax.numpy as jnp
from jax import lax
from jax.experimental import pallas as pl
from jax.experimental.pallas import tpu as pltpu
from jax.experimental.pallas import tpu_sc as plsc

N = 100000
NUM_FEATURES = 9
VOCAB = 128
D = 128
PAIRS = D // 2
N_TILES = 32
ATOMS_PER_TILE = 3136
CHUNK = 64
N_PAD = 100096
FEAT_PAD = 16
TBL_WORDS = NUM_FEATURES * VOCAB * PAIRS + 64
LAST_FULL_CHUNKS = 43
TAIL_ATOMS = 32
FWORDS = CHUNK * FEAT_PAD
OWORDS = CHUNK * D

SCALE = 1.0 / 3.0


def _emit_atom(tbl_v, fbuf, obuf, a, iota, iota2):
    fv = fbuf[pl.ds(a * FEAT_PAD, FEAT_PAD)] * PAIRS
    idx = [
        fv.at[jnp.full((16,), i, jnp.int32)].get(mode="promise_in_bounds")
        + iota
        for i in range(9)
    ]
    obase = a * D
    for c in range(4):
        g = [plsc.bitcast(
                plsc.load_gather(
                    tbl_v.at[pl.ds(i * (VOCAB * PAIRS) + c * 16,
                                   VOCAB * PAIRS)],
                    [idx[i]]),
                jnp.bfloat16) for i in range(9)]
        t01, t23 = g[0] + g[1], g[2] + g[3]
        t45, t67 = g[4] + g[5], g[6] + g[7]
        acc = ((t01 + t23) + (t45 + t67)) + g[8]
        acc32 = plsc.bitcast(acc, jnp.int32)
        lo = plsc.bitcast(lax.shift_left(acc32, 16), jnp.float32) * SCALE
        hi = plsc.bitcast(
            lax.bitwise_and(acc32, jnp.int32(-65536)), jnp.float32) * SCALE
        obuf[pl.ds(obase + c * 16, 16)] = lo
        obuf[pl.ds(obase + 64 + c * 16, 16)] = hi


def _compute_chunk(tbl_v, fbuf, obuf, natoms, iota, iota2):
    def atom_body(k, carry):
        _emit_atom(tbl_v, fbuf, obuf, k * 2, iota, iota2)
        _emit_atom(tbl_v, fbuf, obuf, k * 2 + 1, iota, iota2)
        return carry

    lax.fori_loop(0, natoms // 2, atom_body, 0)


def _body(feats_hbm, tables_hbm, out_hbm, tbl_v, feats_v, out_v, fsem, osem):
    t = lax.axis_index("s") * 2 + lax.axis_index("c")
    pltpu.sync_copy(tables_hbm, tbl_v)

    iota = lax.broadcasted_iota(jnp.int32, (16,), 0)
    iota2 = iota * 2
    base = t * ATOMS_PER_TILE
    nchunks = jnp.where(t == N_TILES - 1, LAST_FULL_CHUNKS,
                        ATOMS_PER_TILE // CHUNK)

    def feats_dma(ci, b):
        row0 = base + ci * CHUNK
        return pltpu.make_async_copy(
            feats_hbm.at[pl.ds(row0 * FEAT_PAD, FWORDS)],
            feats_v.at[pl.ds(b * FWORDS, FWORDS)], fsem.at[b])

    feats_dma(0, 0).start()

    def chunk_body(ci, carry):
        b = lax.rem(ci, 2)
        @pl.when(ci + 1 < nchunks)
        def _pf():
            feats_dma(ci + 1, 1 - b).start()

        @pl.when(ci >= 2)
        def _drain():
            pltpu.make_async_copy(
                out_v.at[pl.ds(b * OWORDS, OWORDS)],
                out_hbm.at[pl.ds((base + (ci - 2) * CHUNK) * D, OWORDS)],
                osem.at[b]).wait()

        feats_dma(ci, b).wait()
        _compute_chunk(tbl_v, feats_v.at[pl.ds(b * FWORDS, FWORDS)],
                       out_v.at[pl.ds(b * OWORDS, OWORDS)],
                       CHUNK, iota, iota2)
        pltpu.async_copy(
            out_v.at[pl.ds(b * OWORDS, OWORDS)],
            out_hbm.at[pl.ds((base + ci * CHUNK) * D, OWORDS)],
            osem.at[b])
        return carry

    lax.fori_loop(0, nchunks, chunk_body, 0)

    for k in (2, 1):
        ci = nchunks - k
        b = lax.rem(ci, 2)
        pltpu.make_async_copy(
            out_v.at[pl.ds(b * OWORDS, OWORDS)],
            out_hbm.at[pl.ds((base + ci * CHUNK) * D, OWORDS)],
            osem.at[b]).wait()

    @pl.when(t == N_TILES - 1)
    def _tail():
        row0 = base + LAST_FULL_CHUNKS * CHUNK
        pltpu.sync_copy(
            feats_hbm.at[pl.ds(row0 * FEAT_PAD, TAIL_ATOMS * FEAT_PAD)],
            feats_v.at[pl.ds(0, TAIL_ATOMS * FEAT_PAD)])
        _compute_chunk(tbl_v, feats_v.at[pl.ds(0, FWORDS)],
                       out_v.at[pl.ds(0, OWORDS)], TAIL_ATOMS, iota, iota2)
        pltpu.sync_copy(
            out_v.at[pl.ds(0, TAIL_ATOMS * D)],
            out_hbm.at[pl.ds(row0 * D, TAIL_ATOMS * D)])


@jax.jit
def kernel(feats, tables):
    feats_p = jnp.pad(
        feats, ((0, N_PAD - N), (0, FEAT_PAD - NUM_FEATURES))).reshape(-1)
    tbl = tables.astype(jnp.bfloat16).reshape(NUM_FEATURES * VOCAB, 2, PAIRS)
    tbl = jnp.swapaxes(tbl, 1, 2)
    tbl = lax.bitcast_convert_type(tbl, jnp.int32).reshape(-1)
    tbl = jnp.pad(tbl, (0, TBL_WORDS - tbl.shape[0]))
    run = pl.kernel(
        _body,
        out_type=jax.ShapeDtypeStruct((N * D,), jnp.float32),
        mesh=plsc.VectorSubcoreMesh(
            core_axis_name="c", subcore_axis_name="s",
            num_cores=2, num_subcores=16),
        compiler_params=pltpu.CompilerParams(needs_layout_passes=False),
        scratch_types=[
            pltpu.VMEM((TBL_WORDS,), jnp.int32),
            pltpu.VMEM((2 * FWORDS,), jnp.int32),
            pltpu.VMEM((2 * OWORDS,), jnp.float32),
            pltpu.SemaphoreType.DMA((2,)),
            pltpu.SemaphoreType.DMA((2,)),
        ],
    )
    return run(feats_p, tbl).reshape(N, D)


# speedup vs baseline: 1.6631x; 1.4877x over previous
import jax
import jax.numpy as jnp
from jax import lax
from jax.experimental import pallas as pl
from jax.experimental.pallas import tpu as pltpu
from jax.experimental.pallas import tpu_sc as plsc

N = 100000
NUM_FEATURES = 9
VOCAB = 128
D = 128
PAIRS = D // 2
N_TILES = 32
ATOMS_PER_TILE = 3136
CHUNK = 64
N_PAD = 100096
FEAT_PAD = 16
TBL_WORDS = NUM_FEATURES * VOCAB * PAIRS + 64
LAST_FULL_CHUNKS = 43
TAIL_ATOMS = 32
FWORDS = CHUNK * FEAT_PAD
OWORDS = CHUNK * D

SCALE = 1.0 / 3.0


def _emit_atom(tbl_v, fbuf, obuf, a, iota, iota2):
    fv = fbuf[pl.ds(a * FEAT_PAD, FEAT_PAD)] * PAIRS
    idx = [
        fv.at[jnp.full((16,), i, jnp.int32)].get(mode="promise_in_bounds")
        + iota
        for i in range(9)
    ]
    obase = a * D
    for c in range(4):
        g = [plsc.bitcast(
                plsc.load_gather(
                    tbl_v.at[pl.ds(i * (VOCAB * PAIRS) + c * 16,
                                   VOCAB * PAIRS)],
                    [idx[i]]),
                jnp.bfloat16) for i in range(9)]
        t01, t23 = g[0] + g[1], g[2] + g[3]
        t45, t67 = g[4] + g[5], g[6] + g[7]
        acc = ((t01 + t23) + (t45 + t67)) + g[8]
        acc32 = plsc.bitcast(acc, jnp.int32)
        lo = plsc.bitcast(lax.shift_left(acc32, 16), jnp.float32) * SCALE
        hi = plsc.bitcast(
            lax.bitwise_and(acc32, jnp.int32(-65536)), jnp.float32) * SCALE
        obuf[pl.ds(obase + c * 16, 16)] = lo
        obuf[pl.ds(obase + 64 + c * 16, 16)] = hi


def _compute_chunk(tbl_v, fbuf, obuf, natoms, iota, iota2):
    @plsc.parallel_loop(0, natoms, step=1, unroll=4)
    def atom_body(a):
        _emit_atom(tbl_v, fbuf, obuf, a, iota, iota2)


def _body(feats_hbm, tables_hbm, out_hbm, tbl_v, feats_v, out_v, fsem, osem):
    t = lax.axis_index("s") * 2 + lax.axis_index("c")
    pltpu.sync_copy(tables_hbm, tbl_v)

    iota = lax.broadcasted_iota(jnp.int32, (16,), 0)
    iota2 = iota * 2
    base = t * ATOMS_PER_TILE
    nchunks = jnp.where(t == N_TILES - 1, LAST_FULL_CHUNKS,
                        ATOMS_PER_TILE // CHUNK)

    def feats_dma(ci, b):
        row0 = base + ci * CHUNK
        return pltpu.make_async_copy(
            feats_hbm.at[pl.ds(row0 * FEAT_PAD, FWORDS)],
            feats_v.at[pl.ds(b * FWORDS, FWORDS)], fsem.at[b])

    feats_dma(0, 0).start()

    def chunk_body(ci, carry):
        b = lax.rem(ci, 2)
        @pl.when(ci + 1 < nchunks)
        def _pf():
            feats_dma(ci + 1, 1 - b).start()

        @pl.when(ci >= 2)
        def _drain():
            pltpu.make_async_copy(
                out_v.at[pl.ds(b * OWORDS, OWORDS)],
                out_hbm.at[pl.ds((base + (ci - 2) * CHUNK) * D, OWORDS)],
                osem.at[b]).wait()

        feats_dma(ci, b).wait()
        _compute_chunk(tbl_v, feats_v.at[pl.ds(b * FWORDS, FWORDS)],
                       out_v.at[pl.ds(b * OWORDS, OWORDS)],
                       CHUNK, iota, iota2)
        pltpu.async_copy(
            out_v.at[pl.ds(b * OWORDS, OWORDS)],
            out_hbm.at[pl.ds((base + ci * CHUNK) * D, OWORDS)],
            osem.at[b])
        return carry

    lax.fori_loop(0, nchunks, chunk_body, 0)

    for k in (2, 1):
        ci = nchunks - k
        b = lax.rem(ci, 2)
        pltpu.make_async_copy(
            out_v.at[pl.ds(b * OWORDS, OWORDS)],
            out_hbm.at[pl.ds((base + ci * CHUNK) * D, OWORDS)],
            osem.at[b]).wait()

    @pl.when(t == N_TILES - 1)
    def _tail():
        row0 = base + LAST_FULL_CHUNKS * CHUNK
        pltpu.sync_copy(
            feats_hbm.at[pl.ds(row0 * FEAT_PAD, TAIL_ATOMS * FEAT_PAD)],
            feats_v.at[pl.ds(0, TAIL_ATOMS * FEAT_PAD)])
        _compute_chunk(tbl_v, feats_v.at[pl.ds(0, FWORDS)],
                       out_v.at[pl.ds(0, OWORDS)], TAIL_ATOMS, iota, iota2)
        pltpu.sync_copy(
            out_v.at[pl.ds(0, TAIL_ATOMS * D)],
            out_hbm.at[pl.ds(row0 * D, TAIL_ATOMS * D)])


@jax.jit
def kernel(feats, tables):
    feats_p = jnp.pad(
        feats, ((0, N_PAD - N), (0, FEAT_PAD - NUM_FEATURES))).reshape(-1)
    tbl = tables.astype(jnp.bfloat16).reshape(NUM_FEATURES * VOCAB, 2, PAIRS)
    tbl = jnp.swapaxes(tbl, 1, 2)
    tbl = lax.bitcast_convert_type(tbl, jnp.int32).reshape(-1)
    tbl = jnp.pad(tbl, (0, TBL_WORDS - tbl.shape[0]))
    run = pl.kernel(
        _body,
        out_type=jax.ShapeDtypeStruct((N * D,), jnp.float32),
        mesh=plsc.VectorSubcoreMesh(
            core_axis_name="c", subcore_axis_name="s",
            num_cores=2, num_subcores=16),
        compiler_params=pltpu.CompilerParams(needs_layout_passes=False),
        scratch_types=[
            pltpu.VMEM((TBL_WORDS,), jnp.int32),
            pltpu.VMEM((2 * FWORDS,), jnp.int32),
            pltpu.VMEM((2 * OWORDS,), jnp.float32),
            pltpu.SemaphoreType.DMA((2,)),
            pltpu.SemaphoreType.DMA((2,)),
        ],
    )
    return run(feats_p, tbl).reshape(N, D)
